# baseline (speedup 1.0000x reference)
import jax
import jax.numpy as jnp
from jax import lax
from jax.experimental import pallas as pl
from jax.experimental.pallas import tpu as pltpu
from jax.experimental.pallas import tpu_sc as plsc

WEEK = 7
DAY = 288
D = 64
LANES = 16
WINDOW = 200
SPLIT = 128


def _build_combined_table(dow_table, tod_table):
    def body(dow_ref, tod_ref, c_ref):
        c_ref[...] = dow_ref[...][:, None, :] + tod_ref[...][None, :, :]

    return pl.pallas_call(
        body,
        out_shape=jax.ShapeDtypeStruct((WEEK, DAY, D), jnp.float32),
    )(dow_table, tod_table)


def _sc_gather(combined, dow, tod, b, t):
    mesh = plsc.VectorSubcoreMesh(core_axis_name="c", subcore_axis_name="s")

    @pl.kernel(
        out_type=jax.ShapeDtypeStruct((b, t, D), jnp.float32),
        mesh=mesh,
        scratch_types=[
            pltpu.VMEM((WINDOW,), jnp.int32),
            pltpu.VMEM_SHARED((WEEK * DAY, D), jnp.float32),
            pltpu.SemaphoreType.DMA,
        ],
        compiler_params=pltpu.CompilerParams(use_tc_tiling_on_sc=False),
    )
    def k(c_hbm, dow_hbm, tod_hbm, out_hbm, idx_ref, c_shared, sem):
        @pl.when(lax.axis_index("s") == 0)
        def _():
            pltpu.sync_copy(c_hbm, c_shared)

        plsc.subcore_barrier()

        def body(dow_v, tod_v, out_v):
            @pl.loop(0, WINDOW - LANES + 1, step=LANES)
            def _(i):
                sl = pl.ds(i, LANES)
                idx_ref[sl] = dow_v.at[0][sl] * DAY + tod_v.at[0][sl]

            tail = pl.ds(WINDOW - LANES, LANES)
            idx_ref[tail] = dow_v.at[0][tail] * DAY + tod_v.at[0][tail]

            c0 = pltpu.async_copy(
                c_shared.at[idx_ref.at[pl.ds(0, SPLIT)]],
                out_v.at[0].at[pl.ds(0, SPLIT)],
                sem,
            )
            c1 = pltpu.async_copy(
                c_shared.at[idx_ref.at[pl.ds(SPLIT, WINDOW - SPLIT)]],
                out_v.at[0].at[pl.ds(SPLIT, WINDOW - SPLIT)],
                sem,
            )
            c0.wait()
            c1.wait()

        pltpu.emit_pipeline(
            body,
            grid=(b,),
            in_specs=[
                pl.BlockSpec((1, WINDOW), index_map=lambda i: (i, 0)),
                pl.BlockSpec((1, WINDOW), index_map=lambda i: (i, 0)),
            ],
            out_specs=[
                pl.BlockSpec((1, WINDOW, D), index_map=lambda i: (i, 0, 0))
            ],
            core_axis_name=("c", "s"),
            dimension_semantics=(pltpu.PARALLEL,),
        )(dow_hbm, tod_hbm, out_hbm)

    return k(combined, dow, tod)


@jax.jit
def kernel(dow, tod, dow_table, tod_table):
    b, t = dow.shape
    combined = _build_combined_table(dow_table, tod_table).reshape(WEEK * DAY, D)
    return _sc_gather(combined, dow.astype(jnp.int32), tod.astype(jnp.int32), b, t)


# speedup vs baseline: 1.3570x; 1.3570x over previous
import jax
import jax.numpy as jnp
from jax import lax
from jax.experimental import pallas as pl
from jax.experimental.pallas import tpu as pltpu
from jax.experimental.pallas import tpu_sc as plsc

WEEK = 7
DAY = 288
D = 64
LANES = 16
WINDOW = 100


def _build_combined_table(dow_table, tod_table):
    def body(dow_ref, tod_ref, c_ref):
        c_ref[...] = dow_ref[...][:, None, :] + tod_ref[...][None, :, :]

    return pl.pallas_call(
        body,
        out_shape=jax.ShapeDtypeStruct((WEEK, DAY, D), jnp.float32),
    )(dow_table, tod_table)


def _sc_gather(combined, dow3, tod3, b, t):
    mesh = plsc.VectorSubcoreMesh(core_axis_name="c", subcore_axis_name="s")
    n_chunks = t // WINDOW

    @pl.kernel(
        out_type=jax.ShapeDtypeStruct((b, t, D), jnp.float32),
        mesh=mesh,
        scratch_types=[
            pltpu.VMEM((WINDOW,), jnp.int32),
            pltpu.VMEM_SHARED((WEEK * DAY, D), jnp.float32),
        ],
        compiler_params=pltpu.CompilerParams(use_tc_tiling_on_sc=False),
    )
    def k(c_hbm, dow_hbm, tod_hbm, out_hbm, idx_ref, c_shared):
        @pl.when(lax.axis_index("s") == 0)
        def _():
            pltpu.sync_copy(c_hbm, c_shared)

        plsc.subcore_barrier()

        def body(dow_v, tod_v, out_v):
            @pl.loop(0, WINDOW - LANES + 1, step=LANES)
            def _(i):
                sl = pl.ds(i, LANES)
                idx_ref[sl] = dow_v.at[0, 0][sl] * DAY + tod_v.at[0, 0][sl]

            tail = pl.ds(WINDOW - LANES, LANES)
            idx_ref[tail] = dow_v.at[0, 0][tail] * DAY + tod_v.at[0, 0][tail]

            pltpu.sync_copy(c_shared.at[idx_ref], out_v.at[0])

        pltpu.emit_pipeline(
            body,
            grid=(b, n_chunks),
            in_specs=[
                pl.BlockSpec((1, 1, WINDOW), index_map=lambda i, j: (i, j, 0)),
                pl.BlockSpec((1, 1, WINDOW), index_map=lambda i, j: (i, j, 0)),
            ],
            out_specs=[
                pl.BlockSpec((1, WINDOW, D), index_map=lambda i, j: (i, j, 0))
            ],
            core_axis_name=("c", "s"),
            dimension_semantics=(pltpu.PARALLEL, pltpu.PARALLEL),
        )(dow_hbm, tod_hbm, out_hbm)

    return k(combined, dow3, tod3)


@jax.jit
def kernel(dow, tod, dow_table, tod_table):
    b, t = dow.shape
    combined = _build_combined_table(dow_table, tod_table).reshape(WEEK * DAY, D)
    dow3 = dow.reshape(b, t // WINDOW, WINDOW).astype(jnp.int32)
    tod3 = tod.reshape(b, t // WINDOW, WINDOW).astype(jnp.int32)
    return _sc_gather(combined, dow3, tod3, b, t)
